# initial kernel scaffold (unmeasured)
import jax
import jax.numpy as jnp
from jax import lax
from jax.experimental import pallas as pl
from jax.experimental.pallas import tpu as pltpu

N_DEV = 4


def kernel(x, router_W, route_idx, expert_W):
    n_tok, d = x.shape
    e_per, _, h = expert_W.shape

    def body(x_ref, rw_ref, idx_ref, ew_ref, out_ref, comm_ref, send_sems, recv_sems):
        my = lax.axis_index("i")
        left = lax.rem(my + N_DEV - 1, N_DEV)
        right = lax.rem(my + 1, N_DEV)

        barrier_sem = pltpu.get_barrier_semaphore()
        pl.semaphore_signal(barrier_sem, inc=1, device_id=(left,),
                            device_id_type=pl.DeviceIdType.MESH)
        pl.semaphore_signal(barrier_sem, inc=1, device_id=(right,),
                            device_id_type=pl.DeviceIdType.MESH)
        pl.semaphore_wait(barrier_sem, 2)

        xv = x_ref[:, :]
        scores = jnp.dot(xv, rw_ref[:, :], preferred_element_type=jnp.float32)
        s_max = jnp.max(scores, axis=-1, keepdims=True)
        p = jnp.exp(scores - s_max)
        probs = p / jnp.sum(p, axis=-1, keepdims=True)
        idx = idx_ref[:, :]
        e0 = idx[:, 0:1]
        e1 = idx[:, 1:2]
        cols = lax.broadcasted_iota(jnp.int32, scores.shape, 1)
        g0 = jnp.sum(jnp.where(cols == e0, probs, 0.0), axis=-1, keepdims=True)
        g1 = jnp.sum(jnp.where(cols == e1, probs, 0.0), axis=-1, keepdims=True)
        gs = g0 + g1
        g0n = g0 / gs
        g1n = g1 / gs

        def stage(w_block, src, accumulate):
            base = src * e_per
            acc = None
            for j in range(e_per):
                ej = base + j
                wj = (jnp.where(e0 == ej, g0n, 0.0)
                      + jnp.where(e1 == ej, g1n, 0.0))
                contrib = jnp.dot(xv * wj, w_block[j],
                                  preferred_element_type=jnp.float32)
                acc = contrib if acc is None else acc + contrib
            if accumulate:
                out_ref[:, :] += acc
            else:
                out_ref[:, :] = acc

        stage(ew_ref, my, False)

        for hop in range(N_DEV - 1):
            src_ref = ew_ref if hop == 0 else comm_ref.at[hop - 1]
            rdma = pltpu.make_async_remote_copy(
                src_ref=src_ref,
                dst_ref=comm_ref.at[hop],
                send_sem=send_sems.at[hop],
                recv_sem=recv_sems.at[hop],
                device_id=(right,),
                device_id_type=pl.DeviceIdType.MESH,
            )
            rdma.start()
            rdma.wait()
            stage(comm_ref.at[hop], lax.rem(my + N_DEV - 1 - hop, N_DEV), True)

    return pl.pallas_call(
        body,
        out_shape=jax.ShapeDtypeStruct((n_tok, h), jnp.float32),
        in_specs=[
            pl.BlockSpec(memory_space=pltpu.VMEM),
            pl.BlockSpec(memory_space=pltpu.VMEM),
            pl.BlockSpec(memory_space=pltpu.VMEM),
            pl.BlockSpec(memory_space=pltpu.VMEM),
        ],
        out_specs=pl.BlockSpec(memory_space=pltpu.VMEM),
        scratch_shapes=[
            pltpu.VMEM((N_DEV - 1, e_per, d, h), jnp.float32),
            pltpu.SemaphoreType.DMA((N_DEV - 1,)),
            pltpu.SemaphoreType.DMA((N_DEV - 1,)),
        ],
        compiler_params=pltpu.CompilerParams(collective_id=0),
    )(x, router_W, route_idx, expert_W)


# baseline (device time: 1091031 ns/iter reference)
import jax
import jax.numpy as jnp
from jax import lax
from jax.experimental import pallas as pl
from jax.experimental.pallas import tpu as pltpu

N_DEV = 4


def kernel(x, router_W, route_idx, expert_W):
    n_tok, d = x.shape
    e_per, _, h = expert_W.shape

    def body(x_ref, rw_ref, idx_ref, ew_hbm, out_ref, all_w,
             wstage, copy_sems, ew_sem, send_sems, recv_sems):
        my = lax.axis_index("i")
        left = lax.rem(my + N_DEV - 1, N_DEV)
        right = lax.rem(my + 1, N_DEV)

        barrier_sem = pltpu.get_barrier_semaphore()
        pl.semaphore_signal(barrier_sem, inc=1, device_id=(left,),
                            device_id_type=pl.DeviceIdType.MESH)
        pl.semaphore_signal(barrier_sem, inc=1, device_id=(right,),
                            device_id_type=pl.DeviceIdType.MESH)
        pl.semaphore_wait(barrier_sem, 2)

        ew_cp = pltpu.make_async_copy(ew_hbm, all_w.at[my], ew_sem)
        ew_cp.start()

        xv = x_ref[:, :]
        scores = jnp.dot(xv, rw_ref[:, :], preferred_element_type=jnp.float32)
        s_max = jnp.max(scores, axis=-1, keepdims=True)
        p = jnp.exp(scores - s_max)
        probs = p / jnp.sum(p, axis=-1, keepdims=True)
        idx = idx_ref[:, :]
        e0 = idx[:, 0:1]
        e1 = idx[:, 1:2]
        cols = lax.broadcasted_iota(jnp.int32, scores.shape, 1)
        g0 = jnp.sum(jnp.where(cols == e0, probs, 0.0), axis=-1, keepdims=True)
        g1 = jnp.sum(jnp.where(cols == e1, probs, 0.0), axis=-1, keepdims=True)
        gs = g0 + g1
        g0n = g0 / gs
        g1n = g1 / gs

        out_ref[:, :] = jnp.zeros((n_tok, h), jnp.float32)
        ew_cp.wait()

        def stage_copy(origin, j, slot):
            return pltpu.make_async_copy(
                all_w.at[origin, j], wstage.at[slot], copy_sems.at[slot])

        def run_stage(origin):
            stage_copy(origin, 0, 0).start()

            def j_body(j, _):
                cur = lax.rem(j, 2)
                nxt = lax.rem(j + 1, 2)

                @pl.when(j + 1 < e_per)
                def _():
                    stage_copy(origin, j + 1, nxt).start()

                stage_copy(origin, j, cur).wait()
                ej = origin * e_per + j
                wj = (jnp.where(e0 == ej, g0n, 0.0)
                      + jnp.where(e1 == ej, g1n, 0.0))
                out_ref[:, :] = out_ref[:, :] + jnp.dot(
                    xv * wj, wstage[cur], preferred_element_type=jnp.float32)
                return 0

            lax.fori_loop(0, e_per, j_body, 0)

        def s_body(s, _):
            origin = lax.rem(my - s + N_DEV, N_DEV)
            sidx = jnp.minimum(s, N_DEV - 2)
            rdma = pltpu.make_async_remote_copy(
                src_ref=all_w.at[origin],
                dst_ref=all_w.at[origin],
                send_sem=send_sems.at[sidx],
                recv_sem=recv_sems.at[sidx],
                device_id=(right,),
                device_id_type=pl.DeviceIdType.MESH,
            )

            @pl.when(s < N_DEV - 1)
            def _():
                rdma.start()

            run_stage(origin)

            @pl.when(s < N_DEV - 1)
            def _():
                rdma.wait()

            return 0

        lax.fori_loop(0, N_DEV, s_body, 0)

    out, _ = pl.pallas_call(
        body,
        out_shape=[
            jax.ShapeDtypeStruct((n_tok, h), jnp.float32),
            jax.ShapeDtypeStruct((N_DEV, e_per, d, h), jnp.float32),
        ],
        in_specs=[
            pl.BlockSpec(memory_space=pltpu.VMEM),
            pl.BlockSpec(memory_space=pltpu.VMEM),
            pl.BlockSpec(memory_space=pltpu.VMEM),
            pl.BlockSpec(memory_space=pltpu.HBM),
        ],
        out_specs=[
            pl.BlockSpec(memory_space=pltpu.VMEM),
            pl.BlockSpec(memory_space=pltpu.HBM),
        ],
        scratch_shapes=[
            pltpu.VMEM((2, d, h), jnp.float32),
            pltpu.SemaphoreType.DMA((2,)),
            pltpu.SemaphoreType.DMA,
            pltpu.SemaphoreType.DMA((N_DEV - 1,)),
            pltpu.SemaphoreType.DMA((N_DEV - 1,)),
        ],
        compiler_params=pltpu.CompilerParams(collective_id=0),
    )(x, router_W, route_idx, expert_W)
    return out


# device time: 326224 ns/iter; 3.3444x vs baseline; 3.3444x over previous
import jax
import jax.numpy as jnp
from jax import lax
from jax.experimental import pallas as pl
from jax.experimental.pallas import tpu as pltpu

N_DEV = 4


def kernel(x, router_W, route_idx, expert_W):
    n_tok, d = x.shape
    e_per, _, h = expert_W.shape

    def body(x_ref, rw_ref, idx_ref, ew_hbm, out_ref, all_w,
             wstage, copy_sems, ew_sem, send_sems, recv_sems):
        my = lax.axis_index("i")
        left = lax.rem(my + N_DEV - 1, N_DEV)
        right = lax.rem(my + 1, N_DEV)

        barrier_sem = pltpu.get_barrier_semaphore()
        pl.semaphore_signal(barrier_sem, inc=1, device_id=(left,),
                            device_id_type=pl.DeviceIdType.MESH)
        pl.semaphore_signal(barrier_sem, inc=1, device_id=(right,),
                            device_id_type=pl.DeviceIdType.MESH)
        pl.semaphore_wait(barrier_sem, 2)

        ew_cp = pltpu.make_async_copy(ew_hbm, all_w.at[my], ew_sem)
        ew_cp.start()

        xv = x_ref[:, :]
        scores = jnp.dot(xv, rw_ref[:, :], preferred_element_type=jnp.float32)
        s_max = jnp.max(scores, axis=-1, keepdims=True)
        p = jnp.exp(scores - s_max)
        probs = p / jnp.sum(p, axis=-1, keepdims=True)
        idx = idx_ref[:, :]
        e0 = idx[:, 0:1]
        e1 = idx[:, 1:2]
        cols = lax.broadcasted_iota(jnp.int32, scores.shape, 1)
        g0 = jnp.sum(jnp.where(cols == e0, probs, 0.0), axis=-1, keepdims=True)
        g1 = jnp.sum(jnp.where(cols == e1, probs, 0.0), axis=-1, keepdims=True)
        gs = g0 + g1
        g0n = g0 / gs
        g1n = g1 / gs
        xvb = xv.astype(jnp.bfloat16)

        out_ref[:, :] = jnp.zeros((n_tok, h), jnp.float32)
        ew_cp.wait()

        def stage_copy(origin, j, slot):
            return pltpu.make_async_copy(
                all_w.at[origin, j], wstage.at[slot], copy_sems.at[slot])

        def run_stage(origin):
            stage_copy(origin, 0, 0).start()

            def j_body(j, _):
                cur = lax.rem(j, 2)
                nxt = lax.rem(j + 1, 2)

                @pl.when(j + 1 < e_per)
                def _():
                    stage_copy(origin, j + 1, nxt).start()

                stage_copy(origin, j, cur).wait()
                ej = origin * e_per + j
                wj = (jnp.where(e0 == ej, g0n, 0.0)
                      + jnp.where(e1 == ej, g1n, 0.0))
                xw = xvb * wj.astype(jnp.bfloat16)
                out_ref[:, :] = out_ref[:, :] + jnp.dot(
                    xw, wstage[cur], preferred_element_type=jnp.float32)
                return 0

            lax.fori_loop(0, e_per, j_body, 0)

        def s_body(s, _):
            origin = lax.rem(my - s + N_DEV, N_DEV)
            sidx = jnp.minimum(s, N_DEV - 2)
            rdma = pltpu.make_async_remote_copy(
                src_ref=all_w.at[origin],
                dst_ref=all_w.at[origin],
                send_sem=send_sems.at[sidx],
                recv_sem=recv_sems.at[sidx],
                device_id=(right,),
                device_id_type=pl.DeviceIdType.MESH,
            )

            @pl.when(s < N_DEV - 1)
            def _():
                rdma.start()

            run_stage(origin)

            @pl.when(s < N_DEV - 1)
            def _():
                rdma.wait()

            return 0

        lax.fori_loop(0, N_DEV, s_body, 0)

    out, _ = pl.pallas_call(
        body,
        out_shape=[
            jax.ShapeDtypeStruct((n_tok, h), jnp.float32),
            jax.ShapeDtypeStruct((N_DEV, e_per, d, h), jnp.bfloat16),
        ],
        in_specs=[
            pl.BlockSpec(memory_space=pltpu.VMEM),
            pl.BlockSpec(memory_space=pltpu.VMEM),
            pl.BlockSpec(memory_space=pltpu.VMEM),
            pl.BlockSpec(memory_space=pltpu.HBM),
        ],
        out_specs=[
            pl.BlockSpec(memory_space=pltpu.VMEM),
            pl.BlockSpec(memory_space=pltpu.HBM),
        ],
        scratch_shapes=[
            pltpu.VMEM((2, d, h), jnp.bfloat16),
            pltpu.SemaphoreType.DMA((2,)),
            pltpu.SemaphoreType.DMA,
            pltpu.SemaphoreType.DMA((N_DEV - 1,)),
            pltpu.SemaphoreType.DMA((N_DEV - 1,)),
        ],
        compiler_params=pltpu.CompilerParams(collective_id=0),
    )(x, router_W, route_idx, expert_W.astype(jnp.bfloat16))
    return out


# device time: 191473 ns/iter; 5.6981x vs baseline; 1.7038x over previous
import jax
import jax.numpy as jnp
from jax import lax
from jax.experimental import pallas as pl
from jax.experimental.pallas import tpu as pltpu

N_DEV = 4


def kernel(x, router_W, route_idx, expert_W):
    n_tok, d = x.shape
    e_per, _, h = expert_W.shape

    half = e_per // 2

    def body(x_ref, rw_ref, idx_ref, ew_hbm, out_ref, all_w,
             wstage, copy_sems, ew_sem,
             send_cw, recv_cw, send_ccw, recv_ccw):
        my = lax.axis_index("i")
        left = lax.rem(my + N_DEV - 1, N_DEV)
        right = lax.rem(my + 1, N_DEV)

        barrier_sem = pltpu.get_barrier_semaphore()
        pl.semaphore_signal(barrier_sem, inc=1, device_id=(left,),
                            device_id_type=pl.DeviceIdType.MESH)
        pl.semaphore_signal(barrier_sem, inc=1, device_id=(right,),
                            device_id_type=pl.DeviceIdType.MESH)
        pl.semaphore_wait(barrier_sem, 2)

        ew_cp = pltpu.make_async_copy(ew_hbm, all_w.at[my], ew_sem)
        ew_cp.start()

        xv = x_ref[:, :]
        scores = jnp.dot(xv, rw_ref[:, :], preferred_element_type=jnp.float32)
        s_max = jnp.max(scores, axis=-1, keepdims=True)
        p = jnp.exp(scores - s_max)
        probs = p / jnp.sum(p, axis=-1, keepdims=True)
        idx = idx_ref[:, :]
        e0 = idx[:, 0:1]
        e1 = idx[:, 1:2]
        cols = lax.broadcasted_iota(jnp.int32, scores.shape, 1)
        g0 = jnp.sum(jnp.where(cols == e0, probs, 0.0), axis=-1, keepdims=True)
        g1 = jnp.sum(jnp.where(cols == e1, probs, 0.0), axis=-1, keepdims=True)
        gs = g0 + g1
        g0n = g0 / gs
        g1n = g1 / gs
        xvb = xv.astype(jnp.bfloat16)

        out_ref[:, :] = jnp.zeros((n_tok, h), jnp.float32)
        ew_cp.wait()

        def stage_copy(origin, j, slot):
            return pltpu.make_async_copy(
                all_w.at[origin, j], wstage.at[slot], copy_sems.at[slot])

        def run_stage(o_cw, o_ccw):
            stage_copy(o_cw, 0, 0).start()

            def j_body(j, _):
                cur = lax.rem(j, 2)
                nxt = lax.rem(j + 1, 2)

                @pl.when(j + 1 < e_per)
                def _():
                    o_nxt = jnp.where(j + 1 < half, o_cw, o_ccw)
                    stage_copy(o_nxt, j + 1, nxt).start()

                o_j = jnp.where(j < half, o_cw, o_ccw)
                stage_copy(o_j, j, cur).wait()
                ej = o_j * e_per + j
                wj = (jnp.where(e0 == ej, g0n, 0.0)
                      + jnp.where(e1 == ej, g1n, 0.0))
                xw = xvb * wj.astype(jnp.bfloat16)
                out_ref[:, :] = out_ref[:, :] + jnp.dot(
                    xw, wstage[cur], preferred_element_type=jnp.float32)
                return 0

            lax.fori_loop(0, e_per, j_body, 0)

        def s_body(s, _):
            o_cw = lax.rem(my - s + N_DEV, N_DEV)
            o_ccw = lax.rem(my + s, N_DEV)
            sidx = jnp.minimum(s, N_DEV - 2)
            rdma_cw = pltpu.make_async_remote_copy(
                src_ref=all_w.at[o_cw, pl.ds(0, half)],
                dst_ref=all_w.at[o_cw, pl.ds(0, half)],
                send_sem=send_cw.at[sidx],
                recv_sem=recv_cw.at[sidx],
                device_id=(right,),
                device_id_type=pl.DeviceIdType.MESH,
            )
            rdma_ccw = pltpu.make_async_remote_copy(
                src_ref=all_w.at[o_ccw, pl.ds(half, half)],
                dst_ref=all_w.at[o_ccw, pl.ds(half, half)],
                send_sem=send_ccw.at[sidx],
                recv_sem=recv_ccw.at[sidx],
                device_id=(left,),
                device_id_type=pl.DeviceIdType.MESH,
            )

            @pl.when(s < N_DEV - 1)
            def _():
                rdma_cw.start()
                rdma_ccw.start()

            run_stage(o_cw, o_ccw)

            @pl.when(s < N_DEV - 1)
            def _():
                rdma_cw.wait()
                rdma_ccw.wait()

            return 0

        lax.fori_loop(0, N_DEV, s_body, 0)

    out, _ = pl.pallas_call(
        body,
        out_shape=[
            jax.ShapeDtypeStruct((n_tok, h), jnp.float32),
            jax.ShapeDtypeStruct((N_DEV, e_per, d, h), jnp.bfloat16),
        ],
        in_specs=[
            pl.BlockSpec(memory_space=pltpu.VMEM),
            pl.BlockSpec(memory_space=pltpu.VMEM),
            pl.BlockSpec(memory_space=pltpu.VMEM),
            pl.BlockSpec(memory_space=pltpu.HBM),
        ],
        out_specs=[
            pl.BlockSpec(memory_space=pltpu.VMEM),
            pl.BlockSpec(memory_space=pltpu.HBM),
        ],
        scratch_shapes=[
            pltpu.VMEM((2, d, h), jnp.bfloat16),
            pltpu.SemaphoreType.DMA((2,)),
            pltpu.SemaphoreType.DMA,
            pltpu.SemaphoreType.DMA((N_DEV - 1,)),
            pltpu.SemaphoreType.DMA((N_DEV - 1,)),
            pltpu.SemaphoreType.DMA((N_DEV - 1,)),
            pltpu.SemaphoreType.DMA((N_DEV - 1,)),
        ],
        compiler_params=pltpu.CompilerParams(collective_id=0),
    )(x, router_W, route_idx, expert_W.astype(jnp.bfloat16))
    return out


# device time: 187642 ns/iter; 5.8144x vs baseline; 1.0204x over previous
import jax
import jax.numpy as jnp
from jax import lax
from jax.experimental import pallas as pl
from jax.experimental.pallas import tpu as pltpu

N_DEV = 4


def kernel(x, router_W, route_idx, expert_W):
    n_tok, d = x.shape
    e_per, _, h = expert_W.shape

    half = e_per // 2

    def body(x_ref, rw_ref, idx_ref, ew_hbm, out_ref, all_w,
             wstage, copy_sems, ew_sem,
             send_cw, recv_cw, send_ccw, recv_ccw):
        my = lax.axis_index("i")
        left = lax.rem(my + N_DEV - 1, N_DEV)
        right = lax.rem(my + 1, N_DEV)

        barrier_sem = pltpu.get_barrier_semaphore()
        pl.semaphore_signal(barrier_sem, inc=1, device_id=(left,),
                            device_id_type=pl.DeviceIdType.MESH)
        pl.semaphore_signal(barrier_sem, inc=1, device_id=(right,),
                            device_id_type=pl.DeviceIdType.MESH)
        pl.semaphore_wait(barrier_sem, 2)

        ew_cp = pltpu.make_async_copy(ew_hbm, all_w.at[my], ew_sem)
        ew_cp.start()
        ew_cp.wait()

        def hop_rdmas(s):
            o_cw = lax.rem(my - s + N_DEV, N_DEV)
            o_ccw = lax.rem(my + s, N_DEV)
            sidx = jnp.minimum(s, N_DEV - 2)
            rdma_cw = pltpu.make_async_remote_copy(
                src_ref=all_w.at[o_cw, pl.ds(0, half)],
                dst_ref=all_w.at[o_cw, pl.ds(0, half)],
                send_sem=send_cw.at[sidx],
                recv_sem=recv_cw.at[sidx],
                device_id=(right,),
                device_id_type=pl.DeviceIdType.MESH,
            )
            rdma_ccw = pltpu.make_async_remote_copy(
                src_ref=all_w.at[o_ccw, pl.ds(half, half)],
                dst_ref=all_w.at[o_ccw, pl.ds(half, half)],
                send_sem=send_ccw.at[sidx],
                recv_sem=recv_ccw.at[sidx],
                device_id=(left,),
                device_id_type=pl.DeviceIdType.MESH,
            )
            return rdma_cw, rdma_ccw

        cw0, ccw0 = hop_rdmas(jnp.int32(0))
        cw0.start()
        ccw0.start()

        xv = x_ref[:, :]
        scores = jnp.dot(xv, rw_ref[:, :], preferred_element_type=jnp.float32)
        s_max = jnp.max(scores, axis=-1, keepdims=True)
        p = jnp.exp(scores - s_max)
        probs = p / jnp.sum(p, axis=-1, keepdims=True)
        idx = idx_ref[:, :]
        e0 = idx[:, 0:1]
        e1 = idx[:, 1:2]
        cols = lax.broadcasted_iota(jnp.int32, scores.shape, 1)
        g0 = jnp.sum(jnp.where(cols == e0, probs, 0.0), axis=-1, keepdims=True)
        g1 = jnp.sum(jnp.where(cols == e1, probs, 0.0), axis=-1, keepdims=True)
        gs = g0 + g1
        g0n = g0 / gs
        g1n = g1 / gs
        xvb = xv.astype(jnp.bfloat16)

        out_ref[:, :] = jnp.zeros((n_tok, h), jnp.float32)

        def stage_copy(origin, j, slot):
            return pltpu.make_async_copy(
                all_w.at[origin, j], wstage.at[slot], copy_sems.at[slot])

        def run_stage(o_cw, o_ccw):
            stage_copy(o_cw, 0, 0).start()

            def j_body(j, _):
                cur = lax.rem(j, 2)
                nxt = lax.rem(j + 1, 2)

                @pl.when(j + 1 < e_per)
                def _():
                    o_nxt = jnp.where(j + 1 < half, o_cw, o_ccw)
                    stage_copy(o_nxt, j + 1, nxt).start()

                o_j = jnp.where(j < half, o_cw, o_ccw)
                stage_copy(o_j, j, cur).wait()
                ej = o_j * e_per + j
                wj = (jnp.where(e0 == ej, g0n, 0.0)
                      + jnp.where(e1 == ej, g1n, 0.0))
                xw = xvb * wj.astype(jnp.bfloat16)
                out_ref[:, :] = out_ref[:, :] + jnp.dot(
                    xw, wstage[cur], preferred_element_type=jnp.float32)
                return 0

            lax.fori_loop(0, e_per, j_body, 0)

        def s_body(s, _):
            o_cw = lax.rem(my - s + N_DEV, N_DEV)
            o_ccw = lax.rem(my + s, N_DEV)
            rdma_cw, rdma_ccw = hop_rdmas(s)

            @pl.when((s >= 1) & (s < N_DEV - 1))
            def _():
                rdma_cw.start()
                rdma_ccw.start()

            run_stage(o_cw, o_ccw)

            @pl.when(s < N_DEV - 1)
            def _():
                rdma_cw.wait()
                rdma_ccw.wait()

            return 0

        lax.fori_loop(0, N_DEV, s_body, 0)

    out, _ = pl.pallas_call(
        body,
        out_shape=[
            jax.ShapeDtypeStruct((n_tok, h), jnp.float32),
            jax.ShapeDtypeStruct((N_DEV, e_per, d, h), jnp.bfloat16),
        ],
        in_specs=[
            pl.BlockSpec(memory_space=pltpu.VMEM),
            pl.BlockSpec(memory_space=pltpu.VMEM),
            pl.BlockSpec(memory_space=pltpu.VMEM),
            pl.BlockSpec(memory_space=pltpu.HBM),
        ],
        out_specs=[
            pl.BlockSpec(memory_space=pltpu.VMEM),
            pl.BlockSpec(memory_space=pltpu.HBM),
        ],
        scratch_shapes=[
            pltpu.VMEM((2, d, h), jnp.bfloat16),
            pltpu.SemaphoreType.DMA((2,)),
            pltpu.SemaphoreType.DMA,
            pltpu.SemaphoreType.DMA((N_DEV - 1,)),
            pltpu.SemaphoreType.DMA((N_DEV - 1,)),
            pltpu.SemaphoreType.DMA((N_DEV - 1,)),
            pltpu.SemaphoreType.DMA((N_DEV - 1,)),
        ],
        compiler_params=pltpu.CompilerParams(collective_id=0),
    )(x, router_W, route_idx, expert_W.astype(jnp.bfloat16))
    return out


# device time: 184254 ns/iter; 5.9213x vs baseline; 1.0184x over previous
import jax
import jax.numpy as jnp
from jax import lax
from jax.experimental import pallas as pl
from jax.experimental.pallas import tpu as pltpu

N_DEV = 4


def kernel(x, router_W, route_idx, expert_W):
    n_tok, d = x.shape
    e_per, _, h = expert_W.shape

    half = e_per // 2

    def body(x_ref, rw_ref, idx_ref, ew_hbm, out_ref, all_w,
             wstage, copy_sems, ew_sem,
             send_cw, recv_cw, send_ccw, recv_ccw):
        my = lax.axis_index("i")
        left = lax.rem(my + N_DEV - 1, N_DEV)
        right = lax.rem(my + 1, N_DEV)

        barrier_sem = pltpu.get_barrier_semaphore()
        pl.semaphore_signal(barrier_sem, inc=1, device_id=(left,),
                            device_id_type=pl.DeviceIdType.MESH)
        pl.semaphore_signal(barrier_sem, inc=1, device_id=(right,),
                            device_id_type=pl.DeviceIdType.MESH)
        pl.semaphore_wait(barrier_sem, 2)

        def hop_rdmas(s):
            o_cw = lax.rem(my - s + N_DEV, N_DEV)
            o_ccw = lax.rem(my + s, N_DEV)
            sidx = jnp.minimum(s, N_DEV - 2)
            rdma_cw = pltpu.make_async_remote_copy(
                src_ref=all_w.at[o_cw, pl.ds(0, half)],
                dst_ref=all_w.at[o_cw, pl.ds(0, half)],
                send_sem=send_cw.at[sidx],
                recv_sem=recv_cw.at[sidx],
                device_id=(right,),
                device_id_type=pl.DeviceIdType.MESH,
            )
            rdma_ccw = pltpu.make_async_remote_copy(
                src_ref=all_w.at[o_ccw, pl.ds(half, half)],
                dst_ref=all_w.at[o_ccw, pl.ds(half, half)],
                send_sem=send_ccw.at[sidx],
                recv_sem=recv_ccw.at[sidx],
                device_id=(left,),
                device_id_type=pl.DeviceIdType.MESH,
            )
            return rdma_cw, rdma_ccw

        cw0 = pltpu.make_async_remote_copy(
            src_ref=ew_hbm.at[pl.ds(0, half)],
            dst_ref=all_w.at[my, pl.ds(0, half)],
            send_sem=send_cw.at[0], recv_sem=recv_cw.at[0],
            device_id=(right,), device_id_type=pl.DeviceIdType.MESH,
        )
        ccw0 = pltpu.make_async_remote_copy(
            src_ref=ew_hbm.at[pl.ds(half, half)],
            dst_ref=all_w.at[my, pl.ds(half, half)],
            send_sem=send_ccw.at[0], recv_sem=recv_ccw.at[0],
            device_id=(left,), device_id_type=pl.DeviceIdType.MESH,
        )
        cw0.start()
        ccw0.start()
        ew_cp = pltpu.make_async_copy(ew_hbm, all_w.at[my], ew_sem)
        ew_cp.start()

        xv = x_ref[:, :]
        scores = jnp.dot(xv, rw_ref[:, :], preferred_element_type=jnp.float32)
        s_max = jnp.max(scores, axis=-1, keepdims=True)
        p = jnp.exp(scores - s_max)
        probs = p / jnp.sum(p, axis=-1, keepdims=True)
        idx = idx_ref[:, :]
        e0 = idx[:, 0:1]
        e1 = idx[:, 1:2]
        cols = lax.broadcasted_iota(jnp.int32, scores.shape, 1)
        g0 = jnp.sum(jnp.where(cols == e0, probs, 0.0), axis=-1, keepdims=True)
        g1 = jnp.sum(jnp.where(cols == e1, probs, 0.0), axis=-1, keepdims=True)
        gs = g0 + g1
        g0n = g0 / gs
        g1n = g1 / gs
        xvb = xv.astype(jnp.bfloat16)

        out_ref[:, :] = jnp.zeros((n_tok, h), jnp.float32)
        ew_cp.wait()

        def stage_copy(origin, j, slot):
            return pltpu.make_async_copy(
                all_w.at[origin, j], wstage.at[slot], copy_sems.at[slot])

        def run_stage(o_cw, o_ccw):
            stage_copy(o_cw, 0, 0).start()

            def j_body(j, _):
                cur = lax.rem(j, 2)
                nxt = lax.rem(j + 1, 2)

                @pl.when(j + 1 < e_per)
                def _():
                    o_nxt = jnp.where(j + 1 < half, o_cw, o_ccw)
                    stage_copy(o_nxt, j + 1, nxt).start()

                o_j = jnp.where(j < half, o_cw, o_ccw)
                stage_copy(o_j, j, cur).wait()
                ej = o_j * e_per + j
                wj = (jnp.where(e0 == ej, g0n, 0.0)
                      + jnp.where(e1 == ej, g1n, 0.0))
                xw = xvb * wj.astype(jnp.bfloat16)
                out_ref[:, :] = out_ref[:, :] + jnp.dot(
                    xw, wstage[cur], preferred_element_type=jnp.float32)
                return 0

            lax.fori_loop(0, e_per, j_body, 0)

        def s_body(s, _):
            o_cw = lax.rem(my - s + N_DEV, N_DEV)
            o_ccw = lax.rem(my + s, N_DEV)
            rdma_cw, rdma_ccw = hop_rdmas(s)

            @pl.when((s >= 1) & (s < N_DEV - 1))
            def _():
                rdma_cw.start()
                rdma_ccw.start()

            run_stage(o_cw, o_ccw)

            @pl.when(s < N_DEV - 1)
            def _():
                rdma_cw.wait()
                rdma_ccw.wait()

            return 0

        lax.fori_loop(0, N_DEV, s_body, 0)

    out, _ = pl.pallas_call(
        body,
        out_shape=[
            jax.ShapeDtypeStruct((n_tok, h), jnp.float32),
            jax.ShapeDtypeStruct((N_DEV, e_per, d, h), jnp.bfloat16),
        ],
        in_specs=[
            pl.BlockSpec(memory_space=pltpu.VMEM),
            pl.BlockSpec(memory_space=pltpu.VMEM),
            pl.BlockSpec(memory_space=pltpu.VMEM),
            pl.BlockSpec(memory_space=pltpu.HBM),
        ],
        out_specs=[
            pl.BlockSpec(memory_space=pltpu.VMEM),
            pl.BlockSpec(memory_space=pltpu.HBM),
        ],
        scratch_shapes=[
            pltpu.VMEM((2, d, h), jnp.bfloat16),
            pltpu.SemaphoreType.DMA((2,)),
            pltpu.SemaphoreType.DMA,
            pltpu.SemaphoreType.DMA((N_DEV - 1,)),
            pltpu.SemaphoreType.DMA((N_DEV - 1,)),
            pltpu.SemaphoreType.DMA((N_DEV - 1,)),
            pltpu.SemaphoreType.DMA((N_DEV - 1,)),
        ],
        compiler_params=pltpu.CompilerParams(collective_id=0),
    )(x, router_W, route_idx, expert_W.astype(jnp.bfloat16))
    return out


# device time: 165867 ns/iter; 6.5777x vs baseline; 1.1109x over previous
import jax
import jax.numpy as jnp
from jax import lax
from jax.experimental import pallas as pl
from jax.experimental.pallas import tpu as pltpu

N_DEV = 4


def kernel(x, router_W, route_idx, expert_W):
    n_tok, d = x.shape
    e_per, _, h = expert_W.shape
    half_e = e_per // 2

    def body(x_ref, rw_ref, idx_ref, ew_hbm, out_ref, all_w,
             wstage, copy_sems, ew_sem, send_sems, recv_sems):
        my = lax.axis_index("i")
        left = lax.rem(my + N_DEV - 1, N_DEV)
        right = lax.rem(my + 1, N_DEV)

        barrier_sem = pltpu.get_barrier_semaphore()
        pl.semaphore_signal(barrier_sem, inc=1, device_id=(left,),
                            device_id_type=pl.DeviceIdType.MESH)
        pl.semaphore_signal(barrier_sem, inc=1, device_id=(right,),
                            device_id_type=pl.DeviceIdType.MESH)
        pl.semaphore_wait(barrier_sem, 2)

        def hop_origin(s, j):
            return jnp.where(j < half_e,
                             lax.rem(my - s + N_DEV, N_DEV),
                             lax.rem(my + s, N_DEV))

        def sub_rdma(s, j, src_in_ew=False):
            o = hop_origin(s, j)
            dst_dev = jnp.where(j < half_e, right, left)
            dirv = jnp.where(j < half_e, 0, 1)
            c = lax.rem(j, half_e)
            sidx = jnp.minimum(s, N_DEV - 2)
            return pltpu.make_async_remote_copy(
                src_ref=ew_hbm.at[j] if src_in_ew else all_w.at[o, j],
                dst_ref=all_w.at[o, j],
                send_sem=send_sems.at[sidx, dirv, c],
                recv_sem=recv_sems.at[sidx, dirv, c],
                device_id=(dst_dev,),
                device_id_type=pl.DeviceIdType.MESH,
            )

        for j in range(e_per):
            sub_rdma(0, j, src_in_ew=True).start()

        ew_cp = pltpu.make_async_copy(ew_hbm, all_w.at[my], ew_sem)
        ew_cp.start()

        xv = x_ref[:, :]
        scores = jnp.dot(xv, rw_ref[:, :], preferred_element_type=jnp.float32)
        s_max = jnp.max(scores, axis=-1, keepdims=True)
        p = jnp.exp(scores - s_max)
        probs = p / jnp.sum(p, axis=-1, keepdims=True)
        idx = idx_ref[:, :]
        e0 = idx[:, 0:1]
        e1 = idx[:, 1:2]
        cols = lax.broadcasted_iota(jnp.int32, scores.shape, 1)
        g0 = jnp.sum(jnp.where(cols == e0, probs, 0.0), axis=-1, keepdims=True)
        g1 = jnp.sum(jnp.where(cols == e1, probs, 0.0), axis=-1, keepdims=True)
        gs = g0 + g1
        g0n = g0 / gs
        g1n = g1 / gs
        xvb = xv.astype(jnp.bfloat16)

        out_ref[:, :] = jnp.zeros((n_tok, h), jnp.float32)
        ew_cp.wait()

        def seq_expert(i):
            return jnp.where(lax.rem(i, 2) == 0, i // 2, half_e + i // 2)

        def stage_copy(o, j, slot):
            return pltpu.make_async_copy(
                all_w.at[o, j], wstage.at[slot], copy_sems.at[slot])

        def compute_expert(s, i):
            ji = seq_expert(i)
            o = hop_origin(s, ji)
            slot = lax.rem(i, 2)
            stage_copy(o, ji, slot).wait()
            ej = o * e_per + ji
            wj = (jnp.where(e0 == ej, g0n, 0.0)
                  + jnp.where(e1 == ej, g1n, 0.0))
            xw = xvb * wj.astype(jnp.bfloat16)
            out_ref[:, :] = out_ref[:, :] + jnp.dot(
                xw, wstage[slot], preferred_element_type=jnp.float32)

        def s_body(s, _):
            def idx_body(i, _):
                @pl.when(i >= 1)
                def _():
                    compute_expert(s, i - 1)

                ji = seq_expert(i)

                @pl.when(s >= 1)
                def _():
                    sub_rdma(s - 1, ji).wait_recv()

                    @pl.when(s < N_DEV - 1)
                    def _():
                        sub_rdma(s, ji).start()

                stage_copy(hop_origin(s, ji), ji, lax.rem(i, 2)).start()
                return 0

            lax.fori_loop(0, e_per, idx_body, 0)
            compute_expert(s, e_per - 1)
            return 0

        lax.fori_loop(0, N_DEV, s_body, 0)

        def drain_body(i, _):
            sub_rdma(i // e_per, lax.rem(i, e_per)).wait_send()
            return 0

        lax.fori_loop(0, (N_DEV - 1) * e_per, drain_body, 0)

    out, _ = pl.pallas_call(
        body,
        out_shape=[
            jax.ShapeDtypeStruct((n_tok, h), jnp.float32),
            jax.ShapeDtypeStruct((N_DEV, e_per, d, h), jnp.bfloat16),
        ],
        in_specs=[
            pl.BlockSpec(memory_space=pltpu.VMEM),
            pl.BlockSpec(memory_space=pltpu.VMEM),
            pl.BlockSpec(memory_space=pltpu.VMEM),
            pl.BlockSpec(memory_space=pltpu.HBM),
        ],
        out_specs=[
            pl.BlockSpec(memory_space=pltpu.VMEM),
            pl.BlockSpec(memory_space=pltpu.HBM),
        ],
        scratch_shapes=[
            pltpu.VMEM((2, d, h), jnp.bfloat16),
            pltpu.SemaphoreType.DMA((2,)),
            pltpu.SemaphoreType.DMA,
            pltpu.SemaphoreType.DMA((N_DEV - 1, 2, half_e)),
            pltpu.SemaphoreType.DMA((N_DEV - 1, 2, half_e)),
        ],
        compiler_params=pltpu.CompilerParams(collective_id=0),
    )(x, router_W, route_idx, expert_W.astype(jnp.bfloat16))
    return out


# device time: 164658 ns/iter; 6.6260x vs baseline; 1.0073x over previous
import jax
import jax.numpy as jnp
from jax import lax
from jax.experimental import pallas as pl
from jax.experimental.pallas import tpu as pltpu

N_DEV = 4


def kernel(x, router_W, route_idx, expert_W):
    n_tok, d = x.shape
    e_per, _, h = expert_W.shape
    half_e = e_per // 2

    def body(x_ref, rw_ref, idx_ref, ew_hbm, out_ref, all_w,
             wstage, copy_sems, ew_sem, send_sems, recv_sems):
        my = lax.axis_index("i")
        left = lax.rem(my + N_DEV - 1, N_DEV)
        right = lax.rem(my + 1, N_DEV)

        barrier_sem = pltpu.get_barrier_semaphore()
        pl.semaphore_signal(barrier_sem, inc=1, device_id=(left,),
                            device_id_type=pl.DeviceIdType.MESH)
        pl.semaphore_signal(barrier_sem, inc=1, device_id=(right,),
                            device_id_type=pl.DeviceIdType.MESH)
        pl.semaphore_wait(barrier_sem, 2)

        def hop_origin(s, j):
            return jnp.where(j < half_e,
                             lax.rem(my - s + N_DEV, N_DEV),
                             lax.rem(my + s, N_DEV))

        def sub_rdma(s, j, src_in_ew=False):
            o = hop_origin(s, j)
            dst_dev = jnp.where(j < half_e, right, left)
            dirv = jnp.where(j < half_e, 0, 1)
            c = lax.rem(j, half_e)
            sidx = jnp.minimum(s, N_DEV - 2)
            return pltpu.make_async_remote_copy(
                src_ref=ew_hbm.at[j] if src_in_ew else all_w.at[o, j],
                dst_ref=all_w.at[o, j],
                send_sem=send_sems.at[sidx, dirv, c],
                recv_sem=recv_sems.at[sidx, dirv, c],
                device_id=(dst_dev,),
                device_id_type=pl.DeviceIdType.MESH,
            )

        for j in range(e_per):
            sub_rdma(0, j, src_in_ew=True).start()

        ew_cp = pltpu.make_async_copy(ew_hbm, all_w.at[my], ew_sem)
        ew_cp.start()

        xv = x_ref[:, :]
        scores = jnp.dot(xv, rw_ref[:, :], preferred_element_type=jnp.float32)
        s_max = jnp.max(scores, axis=-1, keepdims=True)
        p = jnp.exp(scores - s_max)
        probs = p / jnp.sum(p, axis=-1, keepdims=True)
        idx = idx_ref[:, :]
        e0 = idx[:, 0:1]
        e1 = idx[:, 1:2]
        cols = lax.broadcasted_iota(jnp.int32, scores.shape, 1)
        g0 = jnp.sum(jnp.where(cols == e0, probs, 0.0), axis=-1, keepdims=True)
        g1 = jnp.sum(jnp.where(cols == e1, probs, 0.0), axis=-1, keepdims=True)
        gs = g0 + g1
        g0n = g0 / gs
        g1n = g1 / gs
        xvb = xv.astype(jnp.bfloat16)

        out_ref[:, :] = jnp.zeros((n_tok, h), jnp.float32)
        ew_cp.wait()

        def stage_copy(o, j, slot, pos):
            return pltpu.make_async_copy(
                all_w.at[o, j], wstage.at[slot, pos], copy_sems.at[slot, pos])

        def ring_sync(s, j):
            @pl.when(s >= 1)
            def _():
                sub_rdma(s - 1, j).wait_recv()

                @pl.when(s < N_DEV - 1)
                def _():
                    sub_rdma(s, j).start()

        def compute_pair(s, p, slot):
            stage_copy(hop_origin(s, p), p, slot, 0).wait()
            stage_copy(hop_origin(s, half_e + p), half_e + p, slot, 1).wait()
            o_cw = lax.rem(my - s + N_DEV, N_DEV)
            o_ccw = lax.rem(my + s, N_DEV)
            ej_cw = o_cw * e_per + p
            ej_ccw = o_ccw * e_per + half_e + p
            w_cw = (jnp.where(e0 == ej_cw, g0n, 0.0)
                    + jnp.where(e1 == ej_cw, g1n, 0.0))
            w_ccw = (jnp.where(e0 == ej_ccw, g0n, 0.0)
                     + jnp.where(e1 == ej_ccw, g1n, 0.0))
            lhs = jnp.concatenate(
                [xvb * w_cw.astype(jnp.bfloat16),
                 xvb * w_ccw.astype(jnp.bfloat16)], axis=1)
            rhs = wstage[slot].reshape(2 * d, h)
            out_ref[:, :] = out_ref[:, :] + jnp.dot(
                lhs, rhs, preferred_element_type=jnp.float32)

        def s_body(s, _):
            for i in range(e_per):
                p, pos = i // 2, i % 2
                j = p if pos == 0 else half_e + p
                ring_sync(s, j)
                stage_copy(hop_origin(s, j), j, p % 2, pos).start()
                if pos == 1 and p >= 1:
                    compute_pair(s, p - 1, (p - 1) % 2)
            compute_pair(s, e_per // 2 - 1, (e_per // 2 - 1) % 2)
            return 0

        lax.fori_loop(0, N_DEV, s_body, 0)

        def drain_body(i, _):
            sub_rdma(i // e_per, lax.rem(i, e_per)).wait_send()
            return 0

        lax.fori_loop(0, (N_DEV - 1) * e_per, drain_body, 0)

    out, _ = pl.pallas_call(
        body,
        out_shape=[
            jax.ShapeDtypeStruct((n_tok, h), jnp.float32),
            jax.ShapeDtypeStruct((N_DEV, e_per, d, h), jnp.bfloat16),
        ],
        in_specs=[
            pl.BlockSpec(memory_space=pltpu.VMEM),
            pl.BlockSpec(memory_space=pltpu.VMEM),
            pl.BlockSpec(memory_space=pltpu.VMEM),
            pl.BlockSpec(memory_space=pltpu.HBM),
        ],
        out_specs=[
            pl.BlockSpec(memory_space=pltpu.VMEM),
            pl.BlockSpec(memory_space=pltpu.HBM),
        ],
        scratch_shapes=[
            pltpu.VMEM((2, 2, d, h), jnp.bfloat16),
            pltpu.SemaphoreType.DMA((2, 2)),
            pltpu.SemaphoreType.DMA,
            pltpu.SemaphoreType.DMA((N_DEV - 1, 2, half_e)),
            pltpu.SemaphoreType.DMA((N_DEV - 1, 2, half_e)),
        ],
        compiler_params=pltpu.CompilerParams(collective_id=0),
    )(x, router_W, route_idx, expert_W.astype(jnp.bfloat16))
    return out
